# baseline (speedup 1.0000x reference)
import functools

import jax
import jax.numpy as jnp
from jax import lax
from jax.experimental import pallas as pl
from jax.experimental.pallas import tpu as pltpu
from jax.experimental.pallas import tpu_sc as plsc

KTOP = 8
DICT = 100000
CATS = 10
B = 1024
BD = 2048
ND = 49
DPAD = ND * BD
NWIN = DPAD // 128
NEG = -1e30

NC, NS = 2, 16
NWORK = NC * NS
RPW = B // NWORK
NCHUNK = NWIN // 16


def _score_body(x_ref, k_ref, s_ref, wm_ref):
    d = pl.program_id(0)
    x = x_ref[...]
    k = k_ref[...]
    ksq = jnp.sum(k * k, axis=1)
    s = 2.0 * lax.dot_general(
        x, k, (((1,), (1,)), ((), ())), preferred_element_type=jnp.float32
    ) - ksq[None, :]
    col = d * BD + lax.broadcasted_iota(jnp.int32, s.shape, 1)
    s = jnp.where(col < DICT, s, NEG)
    wms = []
    for w in range(BD // 128):
        sw = s[:, w * 128:(w + 1) * 128]
        s_ref[pl.ds(w * B, B), :] = sw
        wms.append(jnp.max(sw, axis=1, keepdims=True))
    wm_ref[0] = jnp.concatenate(wms, axis=1)


def _stage_a(x, kpad):
    return pl.pallas_call(
        _score_body,
        grid=(ND,),
        in_specs=[
            pl.BlockSpec((B, 128), lambda d: (0, 0)),
            pl.BlockSpec((BD, 128), lambda d: (d, 0)),
        ],
        out_specs=[
            pl.BlockSpec(((BD // 128) * B, 128), lambda d: (d, 0)),
            pl.BlockSpec((1, B, BD // 128), lambda d: (d, 0, 0)),
        ],
        out_shape=[
            jax.ShapeDtypeStruct((NWIN * B, 128), jnp.float32),
            jax.ShapeDtypeStruct((ND, B, BD // 128), jnp.float32),
        ],
    )(x, kpad)


_IOTA = lambda: lax.iota(jnp.int32, 16)


def _merge16(Rv, Ri, cv, ci):
    sv, si = plsc.sort_key_val(cv, ci, descending=True)
    svr = lax.rev(sv, (0,))
    sir = lax.rev(si, (0,))
    m = Rv >= svr
    tv = jnp.where(m, Rv, svr)
    ti = jnp.where(m, Ri, sir)
    rv, ri = plsc.sort_key_val(tv, ti, descending=True)
    return rv, ri


def _merge16_pruned(Rv, Ri, cv, ci):
    cmax = lax.reduce_max(cv, (0,))
    return lax.cond(cmax > Rv[7],
                    lambda a, b, c, d: _merge16(a, b, c, d),
                    lambda a, b, c, d: (a, b),
                    Rv, Ri, cv, ci)


def _shift8(v):
    perm = jnp.bitwise_and(_IOTA() + 8, 15)
    dnums = lax.GatherDimensionNumbers(
        offset_dims=(), collapsed_slice_dims=(0,), start_index_map=(0,))
    return lax.gather(v, perm[:, None], dnums, (1,),
                      mode=lax.GatherScatterMode.PROMISE_IN_BOUNDS)


def _sc_body(s2_hbm, wmt_hbm, val_hbm, out_hbm,
             wrow, colb, widx1, widx2, gath, vidx1, vidx2, voff, vrows,
             out_v, sem):
    wid = lax.axis_index("s") * NC + lax.axis_index("c")
    base = wid * RPW

    pltpu.sync_copy(wmt_hbm.at[pl.ds(base, RPW)], wrow)

    def row_top(r):
        def chunk(c, carry):
            Rv, Ri = carry
            cv = wrow[r, pl.ds(c * 16, 16)]
            ci = c * 16 + _IOTA()
            return _merge16_pruned(Rv, Ri, cv, ci)
        Rv0 = jnp.full((16,), NEG, jnp.float32)
        Ri0 = jnp.zeros((16,), jnp.int32)
        _, Ri = lax.fori_loop(0, NCHUNK, chunk, (Rv0, Ri0))
        return Ri

    def pair1(t, widx):
        def body(tt, _):
            ra = 2 * tt
            rb = 2 * tt + 1
            wa = row_top(ra)
            wb = row_top(rb)
            colb[ra, :] = wa * 128
            colb[rb, :] = wb * 128
            ga = wa * B + (base + ra)
            gb = wb * B + (base + rb)
            comb = jnp.where(_IOTA() < 8, ga, _shift8(gb))
            widx[pl.ds((tt - t * 8) * 16, 16)] = comb
            return 0
        lax.fori_loop(t * 8, (t + 1) * 8, body, 0)

    pair1(0, widx1)
    pair1(1, widx2)

    g1 = pltpu.async_copy(s2_hbm.at[widx1], gath.at[pl.ds(0, 128)], sem)
    g2 = pltpu.async_copy(s2_hbm.at[widx2], gath.at[pl.ds(128, 128)], sem)
    g1.wait()
    g2.wait()

    def row_rescan(r):
        crow = colb[r, :]
        Rv = jnp.full((16,), NEG, jnp.float32)
        Ri = jnp.zeros((16,), jnp.int32)
        for c in range(8 * (128 // 16)):
            j = c >> 3
            off = (c & 7) * 16
            cv = gath[r * 8 + j, pl.ds(off, 16)]
            ci = jnp.broadcast_to(crow[j], (16,)) + off + _IOTA()
            Rv, Ri = _merge16_pruned(Rv, Ri, cv, ci)
        return Ri

    def pair2(t, vidx):
        def body(tt, _):
            ra = 2 * tt
            rb = 2 * tt + 1
            ia = row_rescan(ra)
            ib = row_rescan(rb)
            voff[ra, :] = jnp.bitwise_and(ia, 7) * 16
            voff[rb, :] = jnp.bitwise_and(ib, 7) * 16
            comb = jnp.where(_IOTA() < 8, ia, _shift8(ib))
            vidx[pl.ds((tt - t * 8) * 16, 16)] = lax.shift_right_logical(comb, 3)
            return 0
        lax.fori_loop(t * 8, (t + 1) * 8, body, 0)

    pair2(0, vidx1)
    pair2(1, vidx2)

    v1 = pltpu.async_copy(val_hbm.at[vidx1], vrows.at[pl.ds(0, 128)], sem)
    v2 = pltpu.async_copy(val_hbm.at[vidx2], vrows.at[pl.ds(128, 128)], sem)
    v1.wait()
    v2.wait()

    def avg(r, _):
        vo = voff[r, :]
        acc = vrows[r * 8, pl.ds(vo[0], 16)]
        for j in range(1, KTOP):
            acc = acc + vrows[r * 8 + j, pl.ds(vo[j], 16)]
        out_v[r, :] = acc * jnp.float32(1.0 / KTOP)
        return 0
    lax.fori_loop(0, RPW, avg, 0)

    pltpu.sync_copy(out_v, out_hbm.at[pl.ds(base, RPW)])


@functools.partial(
    pl.kernel,
    out_type=jax.ShapeDtypeStruct((B, 16), jnp.float32),
    mesh=plsc.VectorSubcoreMesh(core_axis_name="c", subcore_axis_name="s"),
    compiler_params=pltpu.CompilerParams(needs_layout_passes=False),
    scratch_types=[
        pltpu.VMEM((RPW, NWIN), jnp.float32),
        pltpu.VMEM((RPW, 16), jnp.int32),
        pltpu.VMEM((128,), jnp.int32),
        pltpu.VMEM((128,), jnp.int32),
        pltpu.VMEM((2 * 128, 128), jnp.float32),
        pltpu.VMEM((128,), jnp.int32),
        pltpu.VMEM((128,), jnp.int32),
        pltpu.VMEM((RPW, 16), jnp.int32),
        pltpu.VMEM((2 * 128, 128), jnp.float32),
        pltpu.VMEM((RPW, 16), jnp.float32),
        pltpu.SemaphoreType.DMA,
    ],
)
def _stage_b(s2_hbm, wmt_hbm, val_hbm, out_hbm, *scratch):
    _sc_body(s2_hbm, wmt_hbm, val_hbm, out_hbm, *scratch)


def kernel(x, keys, values):
    scores, wm = _stage_a(x, keys)
    s2 = scores
    wmt = wm.transpose(1, 0, 2).reshape(B, NWIN)
    vpad = jnp.pad(values, ((0, 0), (0, 16 - CATS))).reshape(DICT // 8, 128)
    out16 = _stage_b(s2, wmt, vpad)
    return out16[:, :CATS]


# speedup vs baseline: 1.1214x; 1.1214x over previous
import functools

import jax
import jax.numpy as jnp
from jax import lax
from jax.experimental import pallas as pl
from jax.experimental.pallas import tpu as pltpu
from jax.experimental.pallas import tpu_sc as plsc

KTOP = 8
DICT = 100000
CATS = 10
B = 1024
BD = 1024
ND = 98
DPAD = ND * BD
NWIN = DPAD // 128
NEG = -1e30

NC, NS = 2, 16
NWORK = NC * NS
RPW = B // NWORK
NCHUNK = NWIN // 16


def _score_body(x_ref, k_ref, s_ref, wm_ref):
    d = pl.program_id(0)
    x = x_ref[...]
    k = k_ref[...]
    ksq = jnp.sum(k * k, axis=1)
    s = 2.0 * lax.dot_general(
        x, k, (((1,), (1,)), ((), ())), preferred_element_type=jnp.float32
    ) - ksq[None, :]
    col = d * BD + lax.broadcasted_iota(jnp.int32, s.shape, 1)
    s = jnp.where(col < DICT, s, NEG)
    wms = []
    for w in range(BD // 128):
        sw = s[:, w * 128:(w + 1) * 128]
        s_ref[pl.ds(w * B, B), :] = sw
        wms.append(jnp.max(sw, axis=1, keepdims=True))
    wm_ref[0] = jnp.concatenate(wms, axis=1)


def _stage_a(x, kpad):
    return pl.pallas_call(
        _score_body,
        grid=(ND,),
        in_specs=[
            pl.BlockSpec((B, 128), lambda d: (0, 0)),
            pl.BlockSpec((BD, 128), lambda d: (d, 0)),
        ],
        out_specs=[
            pl.BlockSpec(((BD // 128) * B, 128), lambda d: (d, 0)),
            pl.BlockSpec((1, B, BD // 128), lambda d: (d, 0, 0)),
        ],
        out_shape=[
            jax.ShapeDtypeStruct((NWIN * B, 128), jnp.float32),
            jax.ShapeDtypeStruct((ND, B, BD // 128), jnp.float32),
        ],
    )(x, kpad)


_IOTA = lambda: lax.iota(jnp.int32, 16)


def _merge16(Rv, Ri, cv, ci):
    sv, si = plsc.sort_key_val(cv, ci, descending=True)
    svr = lax.rev(sv, (0,))
    sir = lax.rev(si, (0,))
    m = Rv >= svr
    tv = jnp.where(m, Rv, svr)
    ti = jnp.where(m, Ri, sir)
    rv, ri = plsc.sort_key_val(tv, ti, descending=True)
    return rv, ri


def _shift8(v):
    perm = jnp.bitwise_and(_IOTA() + 8, 15)
    dnums = lax.GatherDimensionNumbers(
        offset_dims=(), collapsed_slice_dims=(0,), start_index_map=(0,))
    return lax.gather(v, perm[:, None], dnums, (1,),
                      mode=lax.GatherScatterMode.PROMISE_IN_BOUNDS)


def _sc_body(s2_hbm, wmt_hbm, val_hbm, out_hbm,
             wrow, colb, widx1, widx2, gath, vidx1, vidx2, voff, vrows,
             out_v, sem):
    wid = lax.axis_index("s") * NC + lax.axis_index("c")
    base = wid * RPW

    pltpu.sync_copy(wmt_hbm.at[pl.ds(base, RPW)], wrow)

    def row_top(r):
        def chunk(c, carry):
            Rv, Ri = carry
            cv = wrow[r, pl.ds(c * 16, 16)]
            ci = c * 16 + _IOTA()
            return _merge16(Rv, Ri, cv, ci)
        Rv0 = jnp.full((16,), NEG, jnp.float32)
        Ri0 = jnp.zeros((16,), jnp.int32)
        _, Ri = lax.fori_loop(0, NCHUNK, chunk, (Rv0, Ri0))
        return Ri

    def pair1(t, widx):
        def body(tt, _):
            ra = 2 * tt
            rb = 2 * tt + 1
            wa = row_top(ra)
            wb = row_top(rb)
            colb[ra, :] = wa * 128
            colb[rb, :] = wb * 128
            ga = wa * B + (base + ra)
            gb = wb * B + (base + rb)
            comb = jnp.where(_IOTA() < 8, ga, _shift8(gb))
            widx[pl.ds((tt - t * 8) * 16, 16)] = comb
            return 0
        lax.fori_loop(t * 8, (t + 1) * 8, body, 0)

    pair1(0, widx1)
    pair1(1, widx2)

    g1 = pltpu.async_copy(s2_hbm.at[widx1], gath.at[pl.ds(0, 128)], sem)
    g2 = pltpu.async_copy(s2_hbm.at[widx2], gath.at[pl.ds(128, 128)], sem)
    g1.wait()
    g2.wait()

    def row_rescan(r):
        crow = colb[r, :]
        Rv = jnp.full((16,), NEG, jnp.float32)
        Ri = jnp.zeros((16,), jnp.int32)
        for c in range(8 * (128 // 16)):
            j = c >> 3
            off = (c & 7) * 16
            cv = gath[r * 8 + j, pl.ds(off, 16)]
            ci = jnp.broadcast_to(crow[j], (16,)) + off + _IOTA()
            Rv, Ri = _merge16(Rv, Ri, cv, ci)
        return Ri

    def pair2(t, vidx):
        def body(tt, _):
            ra = 2 * tt
            rb = 2 * tt + 1
            ia = row_rescan(ra)
            ib = row_rescan(rb)
            voff[ra, :] = jnp.bitwise_and(ia, 7) * 16
            voff[rb, :] = jnp.bitwise_and(ib, 7) * 16
            comb = jnp.where(_IOTA() < 8, ia, _shift8(ib))
            vidx[pl.ds((tt - t * 8) * 16, 16)] = lax.shift_right_logical(comb, 3)
            return 0
        lax.fori_loop(t * 8, (t + 1) * 8, body, 0)

    pair2(0, vidx1)
    pair2(1, vidx2)

    v1 = pltpu.async_copy(val_hbm.at[vidx1], vrows.at[pl.ds(0, 128)], sem)
    v2 = pltpu.async_copy(val_hbm.at[vidx2], vrows.at[pl.ds(128, 128)], sem)
    v1.wait()
    v2.wait()

    def avg(r, _):
        vo = voff[r, :]
        acc = vrows[r * 8, pl.ds(vo[0], 16)]
        for j in range(1, KTOP):
            acc = acc + vrows[r * 8 + j, pl.ds(vo[j], 16)]
        out_v[r, :] = acc * jnp.float32(1.0 / KTOP)
        return 0
    lax.fori_loop(0, RPW, avg, 0)

    pltpu.sync_copy(out_v, out_hbm.at[pl.ds(base, RPW)])


@functools.partial(
    pl.kernel,
    out_type=jax.ShapeDtypeStruct((B, 16), jnp.float32),
    mesh=plsc.VectorSubcoreMesh(core_axis_name="c", subcore_axis_name="s"),
    compiler_params=pltpu.CompilerParams(needs_layout_passes=False),
    scratch_types=[
        pltpu.VMEM((RPW, NWIN), jnp.float32),
        pltpu.VMEM((RPW, 16), jnp.int32),
        pltpu.VMEM((128,), jnp.int32),
        pltpu.VMEM((128,), jnp.int32),
        pltpu.VMEM((2 * 128, 128), jnp.float32),
        pltpu.VMEM((128,), jnp.int32),
        pltpu.VMEM((128,), jnp.int32),
        pltpu.VMEM((RPW, 16), jnp.int32),
        pltpu.VMEM((2 * 128, 128), jnp.float32),
        pltpu.VMEM((RPW, 16), jnp.float32),
        pltpu.SemaphoreType.DMA,
    ],
)
def _stage_b(s2_hbm, wmt_hbm, val_hbm, out_hbm, *scratch):
    _sc_body(s2_hbm, wmt_hbm, val_hbm, out_hbm, *scratch)


def kernel(x, keys, values):
    scores, wm = _stage_a(x, keys)
    s2 = scores
    wmt = wm.transpose(1, 0, 2).reshape(B, NWIN)
    vpad = jnp.pad(values, ((0, 0), (0, 16 - CATS))).reshape(DICT // 8, 128)
    out16 = _stage_b(s2, wmt, vpad)
    return out16[:, :CATS]


# speedup vs baseline: 1.2858x; 1.1467x over previous
import functools

import jax
import jax.numpy as jnp
from jax import lax
from jax.experimental import pallas as pl
from jax.experimental.pallas import tpu as pltpu
from jax.experimental.pallas import tpu_sc as plsc

KTOP = 8
DICT = 100000
CATS = 10
B = 1024
BD = 3584
ND = 28
DPAD = ND * BD
NWIN = DPAD // 128
NEG = -1e30

NC, NS = 2, 16
NWORK = NC * NS
RPW = B // NWORK
NCHUNK = NWIN // 16


def _score_body(x_ref, k_ref, s_ref, wm_ref):
    d = pl.program_id(0)
    x = x_ref[...]
    k = k_ref[...]
    ksq = jnp.sum(k * k, axis=1)
    s = 2.0 * lax.dot_general(
        x, k, (((1,), (1,)), ((), ())), preferred_element_type=jnp.float32
    ) - ksq[None, :]
    col = d * BD + lax.broadcasted_iota(jnp.int32, s.shape, 1)
    s = jnp.where(col < DICT, s, NEG)
    wms = []
    for w in range(BD // 128):
        sw = s[:, w * 128:(w + 1) * 128]
        s_ref[pl.ds(w * B, B), :] = sw
        wms.append(jnp.max(sw, axis=1, keepdims=True))
    wm_ref[0] = jnp.concatenate(wms, axis=1)


def _stage_a(x, kpad):
    return pl.pallas_call(
        _score_body,
        grid=(ND,),
        in_specs=[
            pl.BlockSpec((B, 128), lambda d: (0, 0)),
            pl.BlockSpec((BD, 128), lambda d: (d, 0)),
        ],
        out_specs=[
            pl.BlockSpec(((BD // 128) * B, 128), lambda d: (d, 0)),
            pl.BlockSpec((1, B, BD // 128), lambda d: (d, 0, 0)),
        ],
        out_shape=[
            jax.ShapeDtypeStruct((NWIN * B, 128), jnp.float32),
            jax.ShapeDtypeStruct((ND, B, BD // 128), jnp.float32),
        ],
    )(x, kpad)


_IOTA = lambda: lax.iota(jnp.int32, 16)


def _merge16(Rv, Ri, cv, ci):
    sv, si = plsc.sort_key_val(cv, ci, descending=True)
    svr = lax.rev(sv, (0,))
    sir = lax.rev(si, (0,))
    m = Rv >= svr
    tv = jnp.where(m, Rv, svr)
    ti = jnp.where(m, Ri, sir)
    rv, ri = plsc.sort_key_val(tv, ti, descending=True)
    return rv, ri


def _shift8(v):
    perm = jnp.bitwise_and(_IOTA() + 8, 15)
    dnums = lax.GatherDimensionNumbers(
        offset_dims=(), collapsed_slice_dims=(0,), start_index_map=(0,))
    return lax.gather(v, perm[:, None], dnums, (1,),
                      mode=lax.GatherScatterMode.PROMISE_IN_BOUNDS)


def _sc_body(s2_hbm, wmt_hbm, val_hbm, out_hbm,
             wrow, colb, widx1, widx2, gath, vidx1, vidx2, voff, vrows,
             out_v, sem):
    wid = lax.axis_index("s") * NC + lax.axis_index("c")
    base = wid * RPW

    pltpu.sync_copy(wmt_hbm.at[pl.ds(base, RPW)], wrow)

    def row_top(r):
        def chunk(c, carry):
            Rv, Ri = carry
            cv = wrow[r, pl.ds(c * 16, 16)]
            ci = c * 16 + _IOTA()
            return _merge16(Rv, Ri, cv, ci)
        Rv0 = jnp.full((16,), NEG, jnp.float32)
        Ri0 = jnp.zeros((16,), jnp.int32)
        _, Ri = lax.fori_loop(0, NCHUNK, chunk, (Rv0, Ri0))
        return Ri

    def pair1(t, widx):
        def body(tt, _):
            ra = 2 * tt
            rb = 2 * tt + 1
            wa = row_top(ra)
            wb = row_top(rb)
            colb[ra, :] = wa * 128
            colb[rb, :] = wb * 128
            ga = wa * B + (base + ra)
            gb = wb * B + (base + rb)
            comb = jnp.where(_IOTA() < 8, ga, _shift8(gb))
            widx[pl.ds((tt - t * 8) * 16, 16)] = comb
            return 0
        lax.fori_loop(t * 8, (t + 1) * 8, body, 0)

    pair1(0, widx1)
    pair1(1, widx2)

    g1 = pltpu.async_copy(s2_hbm.at[widx1], gath.at[pl.ds(0, 128)], sem)
    g2 = pltpu.async_copy(s2_hbm.at[widx2], gath.at[pl.ds(128, 128)], sem)
    g1.wait()
    g2.wait()

    def row_rescan(r):
        crow = colb[r, :]
        Rv = jnp.full((16,), NEG, jnp.float32)
        Ri = jnp.zeros((16,), jnp.int32)
        for c in range(8 * (128 // 16)):
            j = c >> 3
            off = (c & 7) * 16
            cv = gath[r * 8 + j, pl.ds(off, 16)]
            ci = jnp.broadcast_to(crow[j], (16,)) + off + _IOTA()
            Rv, Ri = _merge16(Rv, Ri, cv, ci)
        return Ri

    def pair2(t, vidx):
        def body(tt, _):
            ra = 2 * tt
            rb = 2 * tt + 1
            ia = row_rescan(ra)
            ib = row_rescan(rb)
            voff[ra, :] = jnp.bitwise_and(ia, 7) * 16
            voff[rb, :] = jnp.bitwise_and(ib, 7) * 16
            comb = jnp.where(_IOTA() < 8, ia, _shift8(ib))
            vidx[pl.ds((tt - t * 8) * 16, 16)] = lax.shift_right_logical(comb, 3)
            return 0
        lax.fori_loop(t * 8, (t + 1) * 8, body, 0)

    pair2(0, vidx1)
    pair2(1, vidx2)

    v1 = pltpu.async_copy(val_hbm.at[vidx1], vrows.at[pl.ds(0, 128)], sem)
    v2 = pltpu.async_copy(val_hbm.at[vidx2], vrows.at[pl.ds(128, 128)], sem)
    v1.wait()
    v2.wait()

    def avg(r, _):
        vo = voff[r, :]
        acc = vrows[r * 8, pl.ds(vo[0], 16)]
        for j in range(1, KTOP):
            acc = acc + vrows[r * 8 + j, pl.ds(vo[j], 16)]
        out_v[r, :] = acc * jnp.float32(1.0 / KTOP)
        return 0
    lax.fori_loop(0, RPW, avg, 0)

    pltpu.sync_copy(out_v, out_hbm.at[pl.ds(base, RPW)])


@functools.partial(
    pl.kernel,
    out_type=jax.ShapeDtypeStruct((B, 16), jnp.float32),
    mesh=plsc.VectorSubcoreMesh(core_axis_name="c", subcore_axis_name="s"),
    compiler_params=pltpu.CompilerParams(needs_layout_passes=False),
    scratch_types=[
        pltpu.VMEM((RPW, NWIN), jnp.float32),
        pltpu.VMEM((RPW, 16), jnp.int32),
        pltpu.VMEM((128,), jnp.int32),
        pltpu.VMEM((128,), jnp.int32),
        pltpu.VMEM((2 * 128, 128), jnp.float32),
        pltpu.VMEM((128,), jnp.int32),
        pltpu.VMEM((128,), jnp.int32),
        pltpu.VMEM((RPW, 16), jnp.int32),
        pltpu.VMEM((2 * 128, 128), jnp.float32),
        pltpu.VMEM((RPW, 16), jnp.float32),
        pltpu.SemaphoreType.DMA,
    ],
)
def _stage_b(s2_hbm, wmt_hbm, val_hbm, out_hbm, *scratch):
    _sc_body(s2_hbm, wmt_hbm, val_hbm, out_hbm, *scratch)


def kernel(x, keys, values):
    scores, wm = _stage_a(x, keys)
    s2 = scores
    wmt = wm.transpose(1, 0, 2).reshape(B, NWIN)
    vpad = jnp.pad(values, ((0, 0), (0, 16 - CATS))).reshape(DICT // 8, 128)
    out16 = _stage_b(s2, wmt, vpad)
    return out16[:, :CATS]
